# baseline (speedup 1.0000x reference)
import jax
import jax.numpy as jnp
from jax import lax
from jax.experimental import pallas as pl
from jax.experimental.pallas import tpu as pltpu
from jax.experimental.pallas import tpu_sc as plsc

K2 = 16
K1 = 256
BQ = 512
CH = 128
U1 = 8
U23 = 4

_MINI32 = -(2**31)
_MAXI32 = 2**31 - 1


def _le64(th, tl, qh, ql):
    return (th < qh) | ((th == qh) & (tl <= ql))


def kernel(triples, hashes_sorted):
    i32 = jnp.int32
    L = hashes_sorted.shape[0]
    nF = -(-L // K1)
    Lp = nF * K1
    nrow = Lp // K2
    W0 = 1 << ((nF - 1).bit_length() - 1)
    j0 = nF - W0

    hs_f = hashes_sorted[::K1]
    fhi_host = (hs_f >> 32).astype(i32)
    flo_host = ((hs_f & 0xFFFFFFFF) - (1 << 31)).astype(i32)

    hs_d = hashes_sorted[::K2]
    dpad = jnp.full((nrow - hs_d.shape[0],), jnp.int64(2**62))
    hs_d = jnp.concatenate([hs_d, dpad])
    d_rows = jnp.concatenate(
        [(hs_d >> 32).astype(i32).reshape(nF, 16),
         ((hs_d & 0xFFFFFFFF) - (1 << 31)).astype(i32).reshape(nF, 16)],
        axis=1)

    t2 = lax.bitcast_convert_type(hashes_sorted, i32)
    tpad = jnp.full((Lp - L, 2), i32(_MAXI32))
    t_rows = jnp.concatenate([t2, tpad]).reshape(nrow, 32)

    q6 = lax.bitcast_convert_type(
        triples.reshape(-1, 3), i32).reshape(-1, 6)
    N = q6.shape[0]
    info = plsc.get_sparse_core_info()
    NC, NS = info.num_cores, info.num_subcores
    NW = NC * NS
    C = -(-N // (NW * 2 * BQ)) * (2 * BQ)
    nb = C // BQ
    maxoff = N - BQ

    mesh = plsc.VectorSubcoreMesh(core_axis_name="c", subcore_axis_name="s")
    cparams = pltpu.CompilerParams(
        needs_layout_passes=False, use_tc_tiling_on_sc=False)

    def _hash16(qb, qrow):
        z = jnp.zeros((16,), i32)
        s = plsc.load_gather(qb, [qrow, z])
        r = plsc.load_gather(qb, [qrow, z + 2])
        o = plsc.load_gather(qb, [qrow, z + 4])
        qh = (s << 10) | (r >> 11)
        ql = ((r << 21) | o) ^ i32(_MINI32)
        return qh, ql

    def body1(q6_h, fhi_h, flo_h, fpos_h,
              fhi_v, flo_v, qb0, qb1, fb0, fb1,
              sin0, sin1, sout0, sout1):
        wid = lax.axis_index("s") * NC + lax.axis_index("c")
        base = wid * i32(C)
        pltpu.sync_copy(fhi_h, fhi_v)
        pltpu.sync_copy(flo_h, flo_v)
        qbs, fbs = (qb0, qb1), (fb0, fb1)
        sins, souts = (sin0, sin1), (sout0, sout1)
        j0v = jnp.full((16,), i32(j0))
        fh0 = plsc.load_gather(fhi_v, [j0v])
        fl0 = plsc.load_gather(flo_v, [j0v])
        iota = lax.iota(i32, 16)

        def off_of(i):
            return jnp.minimum(base + i * i32(BQ), i32(maxoff))

        def fire_lin(i, p):
            pltpu.async_copy(q6_h.at[pl.ds(off_of(i), BQ)], qbs[p], sins[p])

        def wait_lin(p):
            pltpu.make_async_copy(q6_h.at[pl.ds(0, BQ)], qbs[p],
                                  sins[p]).wait()

        def fire_out(i, p):
            pltpu.async_copy(fbs[p], fpos_h.at[pl.ds(off_of(i), BQ)],
                             souts[p])

        def wait_out(p):
            pltpu.make_async_copy(fbs[p], fpos_h.at[pl.ds(0, BQ)],
                                  souts[p]).wait()

        fire_lin(i32(0), 0)
        fire_lin(i32(1), 1)
        fire_out(i32(0), 0)
        fire_out(i32(1), 1)

        def half(i, p):
            wait_lin(p)
            wait_out(p)

            def phase1(g, c):
                for u in range(U1):
                    v = g * i32(U1) + i32(u)
                    qrow = iota + v * i32(16)
                    qh, ql = _hash16(qbs[p], qrow)
                    pos = jnp.where(_le64(fh0, fl0, qh, ql), i32(j0),
                                    i32(0))
                    w = W0 >> 1
                    while w >= 1:
                        t = pos + w
                        fh = plsc.load_gather(fhi_v, [t])
                        fl = plsc.load_gather(flo_v, [t])
                        pos = jnp.where(_le64(fh, fl, qh, ql), t, pos)
                        w >>= 1
                    fbs[p][pl.ds(v * 16, 16)] = pos
                return c

            lax.fori_loop(i32(0), i32(BQ // 16 // U1), phase1, i32(0))
            fire_out(i, p)
            fire_lin(i + i32(2), p)

        def pair(t, c):
            i = t * i32(2)
            half(i, 0)
            half(i + i32(1), 1)
            return c

        lax.fori_loop(i32(0), i32(nb // 2), pair, i32(0))
        for p in (0, 1):
            wait_out(p)
            wait_lin(p)

    p1 = pl.kernel(
        body1,
        out_type=jax.ShapeDtypeStruct((N,), i32),
        mesh=mesh,
        compiler_params=cparams,
        scratch_types=[
            pltpu.VMEM((nF,), i32), pltpu.VMEM((nF,), i32),
            pltpu.VMEM((BQ, 6), i32), pltpu.VMEM((BQ, 6), i32),
            pltpu.VMEM((BQ,), i32), pltpu.VMEM((BQ,), i32),
            pltpu.SemaphoreType.DMA, pltpu.SemaphoreType.DMA,
            pltpu.SemaphoreType.DMA, pltpu.SemaphoreType.DMA,
        ],
    )

    def make_gather_pass(compute):
        def body(q6_h, idx_h, rows_h, res_h,
                 qb0, qb1, ib0, ib1, gb0, gb1, eb0, eb1,
                 sin0, sin1, sg0, sg1, sout0, sout1):
            wid = lax.axis_index("s") * NC + lax.axis_index("c")
            base = wid * i32(C)
            qbs, ibs = (qb0, qb1), (ib0, ib1)
            gbs, ebs = (gb0, gb1), (eb0, eb1)
            sins, sgs, souts = (sin0, sin1), (sg0, sg1), (sout0, sout1)

            def off_of(i):
                return jnp.minimum(base + i * i32(BQ), i32(maxoff))

            def fire_lin(i, p):
                off = off_of(i)
                pltpu.async_copy(q6_h.at[pl.ds(off, BQ)], qbs[p], sins[p])
                pltpu.async_copy(idx_h.at[pl.ds(off, BQ)], ibs[p], sins[p])

            def wait_lin(p):
                pltpu.make_async_copy(q6_h.at[pl.ds(0, BQ)], qbs[p],
                                      sins[p]).wait()
                pltpu.make_async_copy(idx_h.at[pl.ds(0, BQ)], ibs[p],
                                      sins[p]).wait()

            def fire_gather(p):
                for k in range(BQ // CH):
                    pltpu.async_copy(
                        rows_h.at[ibs[p].at[pl.ds(k * CH, CH)]],
                        gbs[p].at[pl.ds(k * CH, CH)], sgs[p])

            def wait_gather(p):
                for k in range(BQ // CH):
                    pltpu.make_async_copy(
                        rows_h.at[pl.ds(0, CH)],
                        gbs[p].at[pl.ds(k * CH, CH)], sgs[p]).wait()

            def fire_out(i, p):
                pltpu.async_copy(ebs[p], res_h.at[pl.ds(off_of(i), BQ)],
                                 souts[p])

            def wait_out(p):
                pltpu.make_async_copy(ebs[p], res_h.at[pl.ds(0, BQ)],
                                      souts[p]).wait()

            fire_lin(i32(0), 0)
            fire_lin(i32(1), 1)
            wait_lin(0)
            fire_gather(0)
            fire_out(i32(0), 0)
            fire_out(i32(1), 1)

            def half(i, p):
                wait_lin(1 - p)
                fire_gather(1 - p)
                wait_gather(p)
                wait_out(p)
                compute(p, qbs, ibs, gbs, ebs)
                fire_out(i, p)
                fire_lin(i + i32(2), p)

            def pair(t, c):
                i = t * i32(2)
                half(i, 0)
                half(i + i32(1), 1)
                return c

            lax.fori_loop(i32(0), i32(nb // 2), pair, i32(0))
            wait_gather(nb & 1)
            wait_lin((nb + 1) & 1)
            for p in (0, 1):
                wait_out(p)

        return pl.kernel(
            body,
            out_type=jax.ShapeDtypeStruct((N,), i32),
            mesh=mesh,
            compiler_params=cparams,
            scratch_types=[
                pltpu.VMEM((BQ, 6), i32), pltpu.VMEM((BQ, 6), i32),
                pltpu.VMEM((BQ,), i32), pltpu.VMEM((BQ,), i32),
                pltpu.VMEM((BQ, 32), i32), pltpu.VMEM((BQ, 32), i32),
                pltpu.VMEM((BQ,), i32), pltpu.VMEM((BQ,), i32),
                pltpu.SemaphoreType.DMA, pltpu.SemaphoreType.DMA,
                pltpu.SemaphoreType.DMA, pltpu.SemaphoreType.DMA,
                pltpu.SemaphoreType.DMA, pltpu.SemaphoreType.DMA,
            ],
        )

    def compute2(p, qbs, ibs, gbs, ebs):
        it = lax.iota(i32, 16)

        def phase2(g, c):
            for u in range(U23):
                v = g * i32(U23) + i32(u)
                sl = pl.ds(v * 16, 16)
                qrow = it + v * i32(16)
                qh, ql = _hash16(qbs[p], qrow)
                f = ibs[p][sl]
                j = jnp.zeros((16,), i32)
                for w in (8, 4, 2, 1):
                    mid = j + w
                    xh = plsc.load_gather(gbs[p], [qrow, mid])
                    xl = plsc.load_gather(gbs[p], [qrow, mid + 16])
                    j = jnp.where(_le64(xh, xl, qh, ql), mid, j)
                ebs[p][sl] = (f << 4) + j
            return c

        lax.fori_loop(i32(0), i32(BQ // 16 // U23), phase2, i32(0))

    def compute3(p, qbs, ibs, gbs, ebs):
        it = lax.iota(i32, 16)

        def phase3(g, c):
            for u in range(U23):
                v = g * i32(U23) + i32(u)
                sl = pl.ds(v * 16, 16)
                qrow = it + v * i32(16)
                qh, ql = _hash16(qbs[p], qrow)
                j = jnp.zeros((16,), i32)
                for w in (8, 4, 2, 1):
                    mid = j + w
                    m2 = mid + mid
                    xh = plsc.load_gather(gbs[p], [qrow, m2 + 1])
                    xl = plsc.load_gather(gbs[p], [qrow, m2]) ^ i32(_MINI32)
                    j = jnp.where(_le64(xh, xl, qh, ql), mid, j)
                j2 = j + j
                eh = plsc.load_gather(gbs[p], [qrow, j2 + 1])
                el = plsc.load_gather(gbs[p], [qrow, j2]) ^ i32(_MINI32)
                eq = (eh == qh) & (el == ql)
                ebs[p][sl] = jnp.where(eq, i32(0), i32(1))
            return c

        lax.fori_loop(i32(0), i32(BQ // 16 // U23), phase3, i32(0))

    p2 = make_gather_pass(compute2)
    p3 = make_gather_pass(compute3)

    fpos = p1(q6, fhi_host, flo_host)
    trow = p2(q6, fpos, d_rows)
    res = p3(q6, trow, t_rows)
    return (res > 0).reshape(triples.shape[:-1])
